# baseline (speedup 1.0000x reference)
import functools

import jax
import jax.numpy as jnp
from jax import lax
from jax.experimental import pallas as pl
from jax.experimental.pallas import tpu as pltpu

_N = 1024
_B = 32
_EL = 8192
_EG = 32768
_SDIM = 256
_VDIM = 64
_EDIM = 32
_MCOL = 512


def _silu(x):
    return x * jax.nn.sigmoid(x)


def _embed_node_body(x_ref, t_ref, pos8_ref, b_ref,
                     wtma_ref, btma_ref,
                     wam_ref, bam_ref, watm_ref, batm_ref,
                     pos16_ref, s_ref):
    f32 = jnp.float32
    iota_b = lax.broadcasted_iota(jnp.int32, (1, _B), 1)
    oh = (b_ref[...] == iota_b).astype(f32)
    pos8 = pos8_ref[...]
    sums = lax.dot_general(oh, pos8, (((0,), (0,)), ((), ())))
    cnt = jnp.maximum(sums[:, 7:8], 1.0)
    mean8 = sums / cnt
    posc8 = pos8 - jnp.dot(oh, mean8)
    pos16_ref[...] = jnp.concatenate(
        [posc8, jnp.zeros((_N, 120), f32)], axis=1)

    ta = t_ref[...] * wtma_ref[...] + btma_ref[...]
    tnode = jnp.dot(oh, ta)
    s0 = jnp.dot(x_ref[...], wam_ref[...]) + bam_ref[...] + tnode
    s_ref[...] = jnp.dot(s0, watm_ref[...]) + batm_ref[...]


def _embed_edge_body(t_ref, be_ref, ea8_ref,
                     wtmb_ref, btmb_ref, wbm8_ref, bbm_ref, wbtm_ref, bbtm_ref,
                     eg_ref):
    iota_b = lax.broadcasted_iota(jnp.int32, (1, _B), 1)
    ohe = (be_ref[...] == iota_b).astype(jnp.float32)
    tb = t_ref[...] * wtmb_ref[...] + btmb_ref[...]
    tedge = jnp.dot(ohe, tb)
    e0 = jnp.dot(ea8_ref[...], wbm8_ref[...]) + bbm_ref[...] + tedge
    eg = jnp.dot(e0, wbtm_ref[...]) + bbtm_ref[...]
    eg_ref[...] = jnp.concatenate(
        [eg, jnp.zeros((eg.shape[0], 128 - _EDIM), jnp.float32)], axis=1)


def _embed(x, t, pos8, b2d, be2d, ea8, p):
    pos16, s = pl.pallas_call(
        _embed_node_body,
        out_shape=(
            jax.ShapeDtypeStruct((_N, 128), jnp.float32),
            jax.ShapeDtypeStruct((_N, _SDIM), jnp.float32),
        ),
    )(x, t, pos8, b2d,
      p['W_tma'], p['b_tma'].reshape(1, -1),
      p['W_am'], p['b_am'].reshape(1, -1), p['W_atm'], p['b_atm'].reshape(1, -1))
    blk = 4096
    e_g = pl.pallas_call(
        _embed_edge_body,
        grid=(_EG // blk,),
        in_specs=[
            pl.BlockSpec((_B, 1), lambda i: (0, 0)),
            pl.BlockSpec((blk, 1), lambda i: (i, 0)),
            pl.BlockSpec((blk, 8), lambda i: (i, 0)),
            pl.BlockSpec((1, _EDIM), lambda i: (0, 0)),
            pl.BlockSpec((1, _EDIM), lambda i: (0, 0)),
            pl.BlockSpec((8, _EDIM), lambda i: (0, 0)),
            pl.BlockSpec((1, _EDIM), lambda i: (0, 0)),
            pl.BlockSpec((_EDIM, _EDIM), lambda i: (0, 0)),
            pl.BlockSpec((1, _EDIM), lambda i: (0, 0)),
        ],
        out_specs=pl.BlockSpec((blk, 128), lambda i: (i, 0)),
        out_shape=jax.ShapeDtypeStruct((_EG, 128), jnp.float32),
    )(t, be2d, ea8,
      p['W_tmb'], p['b_tmb'].reshape(1, -1),
      jnp.pad(p['W_bm'], ((0, 3), (0, 0))), p['b_bm'].reshape(1, -1),
      p['W_btm'], p['b_btm'].reshape(1, -1))
    return pos16, s, e_g


def _oh_gather8(idx2d, table8):
    iota_n = lax.broadcasted_iota(jnp.int32, (1, _N), 1)
    oh = (idx2d == iota_n).astype(jnp.bfloat16)
    th = table8.astype(jnp.bfloat16)
    tl = (table8 - th.astype(jnp.float32)).astype(jnp.bfloat16)
    return (jnp.dot(oh, th, preferred_element_type=jnp.float32)
            + jnp.dot(oh, tl, preferred_element_type=jnp.float32))


def _msg_body(sg_ref, el_ref, mk_ref, tgt_ref, src_ref, pos_ref,
              w1a_ref, w1b_ref, w1c_ref, b1_ref, wg2_ref, m_ref):
    pos8 = pos_ref[:, 0:8]
    r = (_oh_gather8(tgt_ref[...], pos8)
         - _oh_gather8(src_ref[...], pos8))
    d2 = jnp.sum(r * r, axis=1, keepdims=True)
    d = jnp.sqrt(jnp.clip(d2, 1e-6, None))
    el = el_ref[...][:, :_EDIM] * mk_ref[...]
    pre = (jnp.dot(sg_ref[...], w1a_ref[...]) + jnp.dot(el, w1b_ref[...])
           + d * w1c_ref[...] + b1_ref[...])
    m = _silu(pre)
    gate = jnp.dot(m, wg2_ref[...])
    rn = r[:, 0:3] / d
    be = m.shape[0]
    ones = jnp.ones((be, 1), jnp.float32)
    zero = jnp.zeros((be, _MCOL - _SDIM - 3 * _VDIM - 1), jnp.float32)
    m_ref[...] = jnp.concatenate(
        [m, rn[:, 0:1] * gate, rn[:, 1:2] * gate, rn[:, 2:3] * gate, ones, zero],
        axis=1)


def _msg(sg, el, mk, tgt2d, src2d, pos16, p, nedge):
    blk = 2048
    grid = (nedge // blk,)
    w1a = p['W_g1'][:_SDIM]
    w1b = p['W_g1'][_SDIM:_SDIM + _EDIM]
    w1c = p['W_g1'][_SDIM + _EDIM].reshape(1, -1)
    return pl.pallas_call(
        _msg_body,
        grid=grid,
        in_specs=[
            pl.BlockSpec((blk, _SDIM), lambda i: (i, 0)),
            pl.BlockSpec((blk, 128), lambda i: (i, 0)),
            pl.BlockSpec((blk, 1), lambda i: (i, 0)),
            pl.BlockSpec((blk, 1), lambda i: (i, 0)),
            pl.BlockSpec((blk, 1), lambda i: (i, 0)),
            pl.BlockSpec((_N, 128), lambda i: (0, 0)),
            pl.BlockSpec((_SDIM, _SDIM), lambda i: (0, 0)),
            pl.BlockSpec((_EDIM, _SDIM), lambda i: (0, 0)),
            pl.BlockSpec((1, _SDIM), lambda i: (0, 0)),
            pl.BlockSpec((1, _SDIM), lambda i: (0, 0)),
            pl.BlockSpec((_SDIM, _VDIM), lambda i: (0, 0)),
        ],
        out_specs=pl.BlockSpec((blk, _MCOL), lambda i: (i, 0)),
        out_shape=jax.ShapeDtypeStruct((nedge, _MCOL), jnp.float32),
    )(sg, el, mk, tgt2d, src2d, pos16,
      w1a, w1b, w1c, p['b_g1'].reshape(1, -1), p['W_g2'])


def _comb_body(s_ref, pp_ref, s2_ref):
    ms = pp_ref[...]
    c = jnp.maximum(ms[:, _SDIM + 3 * _VDIM:_SDIM + 3 * _VDIM + 1], 1.0)
    s2_ref[...] = s_ref[...] + ms[:, :_SDIM] / c


def _combine_s(s, partials):
    return pl.pallas_call(
        _comb_body,
        out_shape=jax.ShapeDtypeStruct((_N, _SDIM), jnp.float32),
    )(s, partials)


def _node_body(s2_ref, pl_ref, pg_ref, pos16_ref, b_ref,
               wsh_ref, bsh_ref, wal_ref, bal_ref, wclbd_ref,
               sh_ref, atoms_ref, c16_ref):
    f32 = jnp.float32
    msg = pg_ref[...]
    cg = jnp.maximum(msg[:, _SDIM + 3 * _VDIM:_SDIM + 3 * _VDIM + 1], 1.0)
    s3 = s2_ref[...] + msg[:, :_SDIM] / cg
    msl = pl_ref[...]
    cl = jnp.maximum(msl[:, _SDIM + 3 * _VDIM:_SDIM + 3 * _VDIM + 1], 1.0)
    v3 = msl[:, _SDIM:_SDIM + 3 * _VDIM] / cl + msg[:, _SDIM:_SDIM + 3 * _VDIM] / cg

    sh = _silu(jnp.dot(s3, wsh_ref[...]) + bsh_ref[...])
    sh_ref[...] = sh
    atoms_ref[...] = jnp.dot(sh, wal_ref[...]) + bal_ref[...]

    coords0 = jnp.dot(v3, wclbd_ref[...])
    pos8 = pos16_ref[:, 0:8]
    iota_b = lax.broadcasted_iota(jnp.int32, (1, _B), 1)
    oh = (b_ref[...] == iota_b).astype(f32)
    ones = jnp.ones((_N, 1), f32)
    xcat = jnp.concatenate([coords0, pos8, ones], axis=1)
    sums = lax.dot_general(oh, xcat, (((0,), (0,)), ((), ())))
    cnt = jnp.maximum(sums[:, 16:17], 1.0)
    mean_c = sums[:, 0:8] / cnt
    mean_p = sums[:, 8:16] / cnt
    coords = (coords0 - jnp.dot(oh, mean_c)) + (pos8 - jnp.dot(oh, mean_p))
    c16_ref[...] = coords


def _node_heads(s2, partials_l, partials_g, pos16, b2d, p, wclbd):
    return pl.pallas_call(
        _node_body,
        out_shape=(
            jax.ShapeDtypeStruct((_N, _SDIM), jnp.float32),
            jax.ShapeDtypeStruct((_N, 16), jnp.float32),
            jax.ShapeDtypeStruct((_N, 8), jnp.float32),
        ),
    )(s2, partials_l, partials_g, pos16, b2d,
      p['W_sh'], p['b_sh'].reshape(1, -1), p['W_al'], p['b_al'].reshape(1, -1),
      wclbd)


def _bond_body(shii_ref, shjj_ref, tgt_ref, src_ref, c8_ref, eg_ref,
               whbm_ref, bhbm_ref, wb0a_ref, wb0c_ref, b0_ref,
               wb1_ref, bb1_ref, bonds_ref):
    f = (shii_ref[...] + shjj_ref[...]
         + jnp.dot(eg_ref[...][:, :_EDIM], whbm_ref[...]) + bhbm_ref[...])
    c8 = c8_ref[...]
    dc = _oh_gather8(tgt_ref[...], c8) - _oh_gather8(src_ref[...], c8)
    de = jnp.sqrt(jnp.clip(jnp.sum(dc * dc, axis=1, keepdims=True), 1e-12, None))
    h = _silu(jnp.dot(f, wb0a_ref[...]) + de * wb0c_ref[...] + b0_ref[...])
    bonds_ref[...] = jnp.dot(h, wb1_ref[...]) + bb1_ref[...]


def _bonds(shii, shjj, tgt2d, src2d, c8, e_g, p):
    blk = 2048
    grid = (_EG // blk,)
    wb0a = p['W_b0'][:_SDIM]
    wb0c = p['W_b0'][_SDIM].reshape(1, -1)
    wb1 = jnp.pad(p['W_b1'], ((0, 0), (0, 3)))
    bb1 = jnp.pad(p['b_b1'], (0, 3)).reshape(1, -1)
    out = pl.pallas_call(
        _bond_body,
        grid=grid,
        in_specs=[
            pl.BlockSpec((blk, _SDIM), lambda i: (i, 0)),
            pl.BlockSpec((blk, _SDIM), lambda i: (i, 0)),
            pl.BlockSpec((blk, 1), lambda i: (i, 0)),
            pl.BlockSpec((blk, 1), lambda i: (i, 0)),
            pl.BlockSpec((_N, 8), lambda i: (0, 0)),
            pl.BlockSpec((blk, 128), lambda i: (i, 0)),
            pl.BlockSpec((_EDIM, _SDIM), lambda i: (0, 0)),
            pl.BlockSpec((1, _SDIM), lambda i: (0, 0)),
            pl.BlockSpec((_SDIM, _SDIM), lambda i: (0, 0)),
            pl.BlockSpec((1, _SDIM), lambda i: (0, 0)),
            pl.BlockSpec((1, _SDIM), lambda i: (0, 0)),
            pl.BlockSpec((_SDIM, 8), lambda i: (0, 0)),
            pl.BlockSpec((1, 8), lambda i: (0, 0)),
        ],
        out_specs=pl.BlockSpec((blk, 8), lambda i: (i, 0)),
        out_shape=jax.ShapeDtypeStruct((_EG, 8), jnp.float32),
    )(shii, shjj, tgt2d, src2d, c8, e_g,
      p['W_hbm'], p['b_hbm'].reshape(1, -1), wb0a, wb0c,
      p['b_b0'].reshape(1, -1), wb1, bb1)
    return out[:, :5]


from jax.experimental.pallas import tpu_sc as plsc

_NW = 32


def _sc_mesh():
    return plsc.VectorSubcoreMesh(core_axis_name="c", subcore_axis_name="s")


@functools.lru_cache(maxsize=None)
def _make_sc_gather(nedge, ncols_tuple):
    npair = len(ncols_tuple)
    per_tile = nedge // _NW
    chunk = min(256, per_tile)
    distinct = sorted(set(ncols_tuple))
    buf_of = {nc: i for i, nc in enumerate(distinct)}
    scratch = [pltpu.VMEM((chunk,), jnp.int32)]
    for nc in distinct:
        scratch.append(pltpu.VMEM((chunk, nc), jnp.float32))
    scratch.append(pltpu.SemaphoreType.DMA)

    def body(*refs):
        tables = refs[:npair]
        idxs = refs[npair:2 * npair]
        outs = refs[2 * npair:3 * npair]
        sc = refs[3 * npair:]
        idxv, sem = sc[0], sc[-1]
        wid = lax.axis_index("s") * 2 + lax.axis_index("c")
        base0 = wid * per_tile
        for k in range(npair):
            rowv = sc[1 + buf_of[ncols_tuple[k]]]
            for ch in range(per_tile // chunk):
                base = base0 + ch * chunk
                pltpu.sync_copy(idxs[k].at[pl.ds(base, chunk)], idxv)
                pltpu.async_copy(tables[k].at[idxv], rowv, sem).wait()
                pltpu.sync_copy(rowv, outs[k].at[pl.ds(base, chunk)])

    out_type = [jax.ShapeDtypeStruct((nedge, nc), jnp.float32)
                for nc in ncols_tuple]
    return pl.kernel(body, mesh=_sc_mesh(), out_type=out_type,
                     scratch_types=scratch)


def _sc_gather(pairs):
    nedge = pairs[0][1].shape[0]
    ncols = tuple(int(t.shape[1]) for t, _ in pairs)
    k = _make_sc_gather(nedge, ncols)
    args = [t for t, _ in pairs] + [i for _, i in pairs]
    out = k(*args)
    return list(out) if isinstance(out, (tuple, list)) else [out]


def _unwrap(r):
    return r[0] if isinstance(r, (list, tuple)) else r


def _scatter_body(m_ref, d_ref, out_ref):
    i = pl.program_id(0)

    @pl.when(i == 0)
    def _init():
        out_ref[...] = jnp.zeros_like(out_ref)

    iota_n = lax.broadcasted_iota(jnp.int32, (1, _N), 1)
    oh = (d_ref[...] == iota_n).astype(jnp.bfloat16)
    m = m_ref[...]
    mh = m.astype(jnp.bfloat16)
    ml = (m - mh.astype(jnp.float32)).astype(jnp.bfloat16)
    dn = (((0,), (0,)), ((), ()))
    out_ref[...] += (
        lax.dot_general(oh, mh, dn, preferred_element_type=jnp.float32)
        + lax.dot_general(oh, ml, dn, preferred_element_type=jnp.float32))


def _scatter_stage(m, dst):
    nedge = m.shape[0]
    blk = 2048
    return pl.pallas_call(
        _scatter_body,
        grid=(nedge // blk,),
        in_specs=[
            pl.BlockSpec((blk, _MCOL), lambda i: (i, 0)),
            pl.BlockSpec((blk, 1), lambda i: (i, 0)),
        ],
        out_specs=pl.BlockSpec((_N, _MCOL), lambda i: (0, 0)),
        out_shape=jax.ShapeDtypeStruct((_N, _MCOL), jnp.float32),
    )(m, dst.reshape(nedge, 1))


_KEYS_PER_TILE = (_N * _N) // 16


@functools.lru_cache(maxsize=None)
def _make_sc_match():
    kpt = _KEYS_PER_TILE
    scratch = [
        pltpu.VMEM((kpt,), jnp.int32),
        pltpu.VMEM((4096,), jnp.int32),
        pltpu.VMEM((4096,), jnp.int32),
        pltpu.VMEM((_EL,), jnp.int32),
        pltpu.VMEM((_EL,), jnp.int32),
        pltpu.VMEM((_EL,), jnp.int32),
        pltpu.VMEM((16, 512), jnp.int32),
        pltpu.VMEM((512,), jnp.int32),
        pltpu.VMEM((512,), jnp.float32),
        pltpu.VMEM_SHARED((16, _EL), jnp.int32),
    ]

    def body(srcg, tgtg, srcl, tgtl, z_hbm, wsafe_hbm, maskf_hbm,
             table, eb1, eb2, lsrc, ltgt, wbuf, rbuf, wout, mout, stage):
        c = lax.axis_index("c")
        sid = lax.axis_index("s")
        lo = sid * kpt
        pltpu.sync_copy(z_hbm, table)
        for ch in range(_EG // 4096):
            cb = ch * 4096
            pltpu.sync_copy(srcg.at[pl.ds(cb, 4096)], eb1)
            pltpu.sync_copy(tgtg.at[pl.ds(cb, 4096)], eb2)

            def bfn(i, _, cb=cb):
                key = eb1[pl.ds(i * 16, 16)] * _N + eb2[pl.ds(i * 16, 16)]
                m = (key >= lo) & (key < lo + kpt)
                idx = jnp.where(m, key - lo, 0)
                ids = lax.iota(jnp.int32, 16) + (cb + i * 16 + 1)
                plsc.store_scatter(table, [idx], ids, mask=m)
                return 0

            lax.fori_loop(0, 4096 // 16, bfn, 0)
        pltpu.sync_copy(srcl, lsrc)
        pltpu.sync_copy(tgtl, ltgt)
        pltpu.sync_copy(z_hbm.at[pl.ds(0, _EL)], wbuf)

        def lfn(i, _):
            key = lsrc[pl.ds(i * 16, 16)] * _N + ltgt[pl.ds(i * 16, 16)]
            m = (key >= lo) & (key < lo + kpt)
            idx = jnp.where(m, key - lo, 0)
            vals = plsc.load_gather(table, [idx], mask=m)
            wbuf[pl.ds(i * 16, 16)] = jnp.where(m, vals, 0)
            return 0

        lax.fori_loop(0, _EL // 16, lfn, 0)
        pltpu.sync_copy(wbuf, stage.at[sid])
        plsc.subcore_barrier()
        for j in range(16):
            pltpu.sync_copy(stage.at[j].at[pl.ds(sid * 512, 512)], rbuf.at[j])

        def rfn(k, _):
            acc = rbuf[0, pl.ds(k * 16, 16)]
            for j in range(1, 16):
                acc = jnp.maximum(acc, rbuf[j, pl.ds(k * 16, 16)])
            wout[pl.ds(k * 16, 16)] = jnp.maximum(acc - 1, 0)
            mout[pl.ds(k * 16, 16)] = (acc > 0).astype(jnp.float32)
            return 0

        lax.fori_loop(0, 512 // 16, rfn, 0)

        @pl.when((sid < 8) == (c == 0))
        def _wr():
            pltpu.sync_copy(wout, wsafe_hbm.at[pl.ds(sid * 512, 512)])
            pltpu.sync_copy(mout, maskf_hbm.at[pl.ds(sid * 512, 512)])

    return pl.kernel(
        body, mesh=_sc_mesh(),
        out_type=[jax.ShapeDtypeStruct((_EL,), jnp.int32),
                  jax.ShapeDtypeStruct((_EL,), jnp.float32)],
        scratch_types=scratch,
        compiler_params=pltpu.CompilerParams(needs_layout_passes=False))


def _match_stage(src_g, tgt_g, src_l, tgt_l):
    z = jnp.zeros((_KEYS_PER_TILE,), jnp.int32)
    return _make_sc_match()(src_g, tgt_g, src_l, tgt_l, z)


def kernel(x, t, pos, edge_index_local, edge_index_global, edge_attr_global,
           batch, batch_edge_global, params):
    p = params
    src_l, tgt_l = edge_index_local[0], edge_index_local[1]
    src_g, tgt_g = edge_index_global[0], edge_index_global[1]

    pos8 = jnp.concatenate(
        [pos, jnp.zeros((_N, 4), jnp.float32), jnp.ones((_N, 1), jnp.float32)],
        axis=1)
    b2d = batch.astype(jnp.int32).reshape(_N, 1)
    be2d = batch_edge_global.astype(jnp.int32).reshape(_EG, 1)
    ea8 = jnp.pad(edge_attr_global, ((0, 0), (0, 3)))

    pos16, s, e_g = _embed(x, t, pos8, b2d, be2d, ea8, p)

    src_l2d = src_l.reshape(_EL, 1)
    tgt_l2d = tgt_l.reshape(_EL, 1)
    src_g2d = src_g.reshape(_EG, 1)
    tgt_g2d = tgt_g.reshape(_EG, 1)

    wsafe, maskf = _match_stage(src_g, tgt_g, src_l, tgt_l)
    sg_l, e_l = _sc_gather([(s, src_l), (e_g, wsafe)])
    m_l = _msg(sg_l, e_l, maskf.reshape(_EL, 1), tgt_l2d, src_l2d, pos16,
               p, _EL)
    part_l = _scatter_stage(m_l, tgt_l)
    s2 = _combine_s(s, part_l)

    sg_g = _sc_gather([(s2, src_g)])[0]
    ones_mask = jnp.ones((_EG, 1), jnp.float32)
    m_g = _msg(sg_g, e_g, ones_mask, tgt_g2d, src_g2d, pos16, p, _EG)
    part_g = _scatter_stage(m_g, tgt_g)

    wclbd = jnp.zeros((3 * _VDIM, 8), jnp.float32)
    for k in range(3):
        wclbd = wclbd.at[k * _VDIM:(k + 1) * _VDIM, k].set(p['W_cl'][:, 0])
    sh, atoms, c8 = _node_heads(s2, part_l, part_g, pos16, b2d, p, wclbd)

    shii, shjj = _sc_gather([(sh, tgt_g), (sh, src_g)])
    bonds = _bonds(shii, shjj, tgt_g2d, src_g2d, c8, e_g, p)

    coords = c8[:, :3]
    return coords, atoms, bonds


# speedup vs baseline: 1.0028x; 1.0028x over previous
import functools

import jax
import jax.numpy as jnp
from jax import lax
from jax.experimental import pallas as pl
from jax.experimental.pallas import tpu as pltpu

_N = 1024
_B = 32
_EL = 8192
_EG = 32768
_SDIM = 256
_VDIM = 64
_EDIM = 32
_MCOL = 512


def _silu(x):
    return x * jax.nn.sigmoid(x)


def _embed_node_body(x_ref, t_ref, pos8_ref, b_ref,
                     wtma_ref, btma_ref,
                     wam_ref, bam_ref, watm_ref, batm_ref,
                     pos16_ref, s_ref):
    f32 = jnp.float32
    iota_b = lax.broadcasted_iota(jnp.int32, (1, _B), 1)
    oh = (b_ref[...] == iota_b).astype(f32)
    pos8 = pos8_ref[...]
    sums = lax.dot_general(oh, pos8, (((0,), (0,)), ((), ())))
    cnt = jnp.maximum(sums[:, 7:8], 1.0)
    mean8 = sums / cnt
    posc8 = pos8 - jnp.dot(oh, mean8)
    pos16_ref[...] = jnp.concatenate(
        [posc8, jnp.zeros((_N, 120), f32)], axis=1)

    ta = t_ref[...] * wtma_ref[...] + btma_ref[...]
    tnode = jnp.dot(oh, ta)
    s0 = jnp.dot(x_ref[...], wam_ref[...]) + bam_ref[...] + tnode
    s_ref[...] = jnp.dot(s0, watm_ref[...]) + batm_ref[...]


def _embed_edge_body(t_ref, be_ref, ea8_ref,
                     wtmb_ref, btmb_ref, wbm8_ref, bbm_ref, wbtm_ref, bbtm_ref,
                     eg_ref):
    iota_b = lax.broadcasted_iota(jnp.int32, (1, _B), 1)
    ohe = (be_ref[...] == iota_b).astype(jnp.float32)
    tb = t_ref[...] * wtmb_ref[...] + btmb_ref[...]
    tedge = jnp.dot(ohe, tb)
    e0 = jnp.dot(ea8_ref[...], wbm8_ref[...]) + bbm_ref[...] + tedge
    eg = jnp.dot(e0, wbtm_ref[...]) + bbtm_ref[...]
    eg_ref[...] = jnp.concatenate(
        [eg, jnp.zeros((eg.shape[0], 128 - _EDIM), jnp.float32)], axis=1)


def _embed(x, t, pos8, b2d, be2d, ea8, p):
    pos16, s = pl.pallas_call(
        _embed_node_body,
        out_shape=(
            jax.ShapeDtypeStruct((_N, 128), jnp.float32),
            jax.ShapeDtypeStruct((_N, _SDIM), jnp.float32),
        ),
    )(x, t, pos8, b2d,
      p['W_tma'], p['b_tma'].reshape(1, -1),
      p['W_am'], p['b_am'].reshape(1, -1), p['W_atm'], p['b_atm'].reshape(1, -1))
    blk = 4096
    e_g = pl.pallas_call(
        _embed_edge_body,
        grid=(_EG // blk,),
        in_specs=[
            pl.BlockSpec((_B, 1), lambda i: (0, 0)),
            pl.BlockSpec((blk, 1), lambda i: (i, 0)),
            pl.BlockSpec((blk, 8), lambda i: (i, 0)),
            pl.BlockSpec((1, _EDIM), lambda i: (0, 0)),
            pl.BlockSpec((1, _EDIM), lambda i: (0, 0)),
            pl.BlockSpec((8, _EDIM), lambda i: (0, 0)),
            pl.BlockSpec((1, _EDIM), lambda i: (0, 0)),
            pl.BlockSpec((_EDIM, _EDIM), lambda i: (0, 0)),
            pl.BlockSpec((1, _EDIM), lambda i: (0, 0)),
        ],
        out_specs=pl.BlockSpec((blk, 128), lambda i: (i, 0)),
        out_shape=jax.ShapeDtypeStruct((_EG, 128), jnp.float32),
    )(t, be2d, ea8,
      p['W_tmb'], p['b_tmb'].reshape(1, -1),
      jnp.pad(p['W_bm'], ((0, 3), (0, 0))), p['b_bm'].reshape(1, -1),
      p['W_btm'], p['b_btm'].reshape(1, -1))
    return pos16, s, e_g


def _oh_gather8(idx2d, table8):
    iota_n = lax.broadcasted_iota(jnp.int32, (1, _N), 1)
    oh = (idx2d == iota_n).astype(jnp.bfloat16)
    th = table8.astype(jnp.bfloat16)
    tl = (table8 - th.astype(jnp.float32)).astype(jnp.bfloat16)
    return (jnp.dot(oh, th, preferred_element_type=jnp.float32)
            + jnp.dot(oh, tl, preferred_element_type=jnp.float32))


def _msg_body(sg_ref, el_ref, mk_ref, tgt_ref, src_ref, pos_ref,
              w1a_ref, w1b_ref, w1c_ref, b1_ref, wg2_ref, m_ref):
    pos8 = pos_ref[:, 0:8]
    r = (_oh_gather8(tgt_ref[...], pos8)
         - _oh_gather8(src_ref[...], pos8))
    d2 = jnp.sum(r * r, axis=1, keepdims=True)
    d = jnp.sqrt(jnp.clip(d2, 1e-6, None))
    el = el_ref[...][:, :_EDIM] * mk_ref[...]
    pre = (jnp.dot(sg_ref[...], w1a_ref[...]) + jnp.dot(el, w1b_ref[...])
           + d * w1c_ref[...] + b1_ref[...])
    m = _silu(pre)
    gate = jnp.dot(m, wg2_ref[...])
    rn = r[:, 0:3] / d
    be = m.shape[0]
    ones = jnp.ones((be, 1), jnp.float32)
    zero = jnp.zeros((be, _MCOL - _SDIM - 3 * _VDIM - 1), jnp.float32)
    m_ref[...] = jnp.concatenate(
        [m, rn[:, 0:1] * gate, rn[:, 1:2] * gate, rn[:, 2:3] * gate, ones, zero],
        axis=1)


def _msg(sg, el, mk, tgt2d, src2d, pos16, p, nedge):
    blk = 2048
    grid = (nedge // blk,)
    w1a = p['W_g1'][:_SDIM]
    w1b = p['W_g1'][_SDIM:_SDIM + _EDIM]
    w1c = p['W_g1'][_SDIM + _EDIM].reshape(1, -1)
    return pl.pallas_call(
        _msg_body,
        grid=grid,
        in_specs=[
            pl.BlockSpec((blk, _SDIM), lambda i: (i, 0)),
            pl.BlockSpec((blk, 128), lambda i: (i, 0)),
            pl.BlockSpec((blk, 1), lambda i: (i, 0)),
            pl.BlockSpec((blk, 1), lambda i: (i, 0)),
            pl.BlockSpec((blk, 1), lambda i: (i, 0)),
            pl.BlockSpec((_N, 128), lambda i: (0, 0)),
            pl.BlockSpec((_SDIM, _SDIM), lambda i: (0, 0)),
            pl.BlockSpec((_EDIM, _SDIM), lambda i: (0, 0)),
            pl.BlockSpec((1, _SDIM), lambda i: (0, 0)),
            pl.BlockSpec((1, _SDIM), lambda i: (0, 0)),
            pl.BlockSpec((_SDIM, _VDIM), lambda i: (0, 0)),
        ],
        out_specs=pl.BlockSpec((blk, _MCOL), lambda i: (i, 0)),
        out_shape=jax.ShapeDtypeStruct((nedge, _MCOL), jnp.float32),
    )(sg, el, mk, tgt2d, src2d, pos16,
      w1a, w1b, w1c, p['b_g1'].reshape(1, -1), p['W_g2'])


def _comb_body(s_ref, pp_ref, s2_ref):
    ms = pp_ref[...]
    c = jnp.maximum(ms[:, _SDIM + 3 * _VDIM:_SDIM + 3 * _VDIM + 1], 1.0)
    s2_ref[...] = s_ref[...] + ms[:, :_SDIM] / c


def _combine_s(s, partials):
    return pl.pallas_call(
        _comb_body,
        out_shape=jax.ShapeDtypeStruct((_N, _SDIM), jnp.float32),
    )(s, partials)


def _node_body(s2_ref, pl_ref, pg_ref, pos16_ref, b_ref,
               wsh_ref, bsh_ref, wal_ref, bal_ref, wclbd_ref,
               sh_ref, atoms_ref, c16_ref):
    f32 = jnp.float32
    msg = pg_ref[...]
    cg = jnp.maximum(msg[:, _SDIM + 3 * _VDIM:_SDIM + 3 * _VDIM + 1], 1.0)
    s3 = s2_ref[...] + msg[:, :_SDIM] / cg
    msl = pl_ref[...]
    cl = jnp.maximum(msl[:, _SDIM + 3 * _VDIM:_SDIM + 3 * _VDIM + 1], 1.0)
    v3 = msl[:, _SDIM:_SDIM + 3 * _VDIM] / cl + msg[:, _SDIM:_SDIM + 3 * _VDIM] / cg

    sh = _silu(jnp.dot(s3, wsh_ref[...]) + bsh_ref[...])
    sh_ref[...] = sh
    atoms_ref[...] = jnp.dot(sh, wal_ref[...]) + bal_ref[...]

    coords0 = jnp.dot(v3, wclbd_ref[...])
    pos8 = pos16_ref[:, 0:8]
    iota_b = lax.broadcasted_iota(jnp.int32, (1, _B), 1)
    oh = (b_ref[...] == iota_b).astype(f32)
    ones = jnp.ones((_N, 1), f32)
    xcat = jnp.concatenate([coords0, pos8, ones], axis=1)
    sums = lax.dot_general(oh, xcat, (((0,), (0,)), ((), ())))
    cnt = jnp.maximum(sums[:, 16:17], 1.0)
    mean_c = sums[:, 0:8] / cnt
    mean_p = sums[:, 8:16] / cnt
    coords = (coords0 - jnp.dot(oh, mean_c)) + (pos8 - jnp.dot(oh, mean_p))
    c16_ref[...] = coords


def _node_heads(s2, partials_l, partials_g, pos16, b2d, p, wclbd):
    return pl.pallas_call(
        _node_body,
        out_shape=(
            jax.ShapeDtypeStruct((_N, _SDIM), jnp.float32),
            jax.ShapeDtypeStruct((_N, 16), jnp.float32),
            jax.ShapeDtypeStruct((_N, 8), jnp.float32),
        ),
    )(s2, partials_l, partials_g, pos16, b2d,
      p['W_sh'], p['b_sh'].reshape(1, -1), p['W_al'], p['b_al'].reshape(1, -1),
      wclbd)


def _bond_body(shii_ref, shjj_ref, tgt_ref, src_ref, c8_ref, eg_ref,
               whbm_ref, bhbm_ref, wb0a_ref, wb0c_ref, b0_ref,
               wb1_ref, bb1_ref, bonds_ref):
    f = (shii_ref[...] + shjj_ref[...]
         + jnp.dot(eg_ref[...][:, :_EDIM], whbm_ref[...]) + bhbm_ref[...])
    c8 = c8_ref[...]
    dc = _oh_gather8(tgt_ref[...], c8) - _oh_gather8(src_ref[...], c8)
    de = jnp.sqrt(jnp.clip(jnp.sum(dc * dc, axis=1, keepdims=True), 1e-12, None))
    h = _silu(jnp.dot(f, wb0a_ref[...]) + de * wb0c_ref[...] + b0_ref[...])
    bonds_ref[...] = jnp.dot(h, wb1_ref[...]) + bb1_ref[...]


def _bonds(shii, shjj, tgt2d, src2d, c8, e_g, p):
    blk = 2048
    grid = (_EG // blk,)
    wb0a = p['W_b0'][:_SDIM]
    wb0c = p['W_b0'][_SDIM].reshape(1, -1)
    wb1 = jnp.pad(p['W_b1'], ((0, 0), (0, 3)))
    bb1 = jnp.pad(p['b_b1'], (0, 3)).reshape(1, -1)
    out = pl.pallas_call(
        _bond_body,
        grid=grid,
        in_specs=[
            pl.BlockSpec((blk, _SDIM), lambda i: (i, 0)),
            pl.BlockSpec((blk, _SDIM), lambda i: (i, 0)),
            pl.BlockSpec((blk, 1), lambda i: (i, 0)),
            pl.BlockSpec((blk, 1), lambda i: (i, 0)),
            pl.BlockSpec((_N, 8), lambda i: (0, 0)),
            pl.BlockSpec((blk, 128), lambda i: (i, 0)),
            pl.BlockSpec((_EDIM, _SDIM), lambda i: (0, 0)),
            pl.BlockSpec((1, _SDIM), lambda i: (0, 0)),
            pl.BlockSpec((_SDIM, _SDIM), lambda i: (0, 0)),
            pl.BlockSpec((1, _SDIM), lambda i: (0, 0)),
            pl.BlockSpec((1, _SDIM), lambda i: (0, 0)),
            pl.BlockSpec((_SDIM, 8), lambda i: (0, 0)),
            pl.BlockSpec((1, 8), lambda i: (0, 0)),
        ],
        out_specs=pl.BlockSpec((blk, 8), lambda i: (i, 0)),
        out_shape=jax.ShapeDtypeStruct((_EG, 8), jnp.float32),
    )(shii, shjj, tgt2d, src2d, c8, e_g,
      p['W_hbm'], p['b_hbm'].reshape(1, -1), wb0a, wb0c,
      p['b_b0'].reshape(1, -1), wb1, bb1)
    return out[:, :5]


from jax.experimental.pallas import tpu_sc as plsc

_NW = 32


def _sc_mesh():
    return plsc.VectorSubcoreMesh(core_axis_name="c", subcore_axis_name="s")


@functools.lru_cache(maxsize=None)
def _make_sc_gather(nedge, ncols_tuple):
    npair = len(ncols_tuple)
    per_tile = nedge // _NW
    chunk = min(256, per_tile)
    distinct = sorted(set(ncols_tuple))
    buf_of = {nc: i for i, nc in enumerate(distinct)}
    scratch = [pltpu.VMEM((chunk,), jnp.int32)]
    for nc in distinct:
        scratch.append(pltpu.VMEM((chunk, nc), jnp.float32))
    scratch.append(pltpu.SemaphoreType.DMA)

    def body(*refs):
        tables = refs[:npair]
        idxs = refs[npair:2 * npair]
        outs = refs[2 * npair:3 * npair]
        sc = refs[3 * npair:]
        idxv, sem = sc[0], sc[-1]
        wid = lax.axis_index("s") * 2 + lax.axis_index("c")
        base0 = wid * per_tile
        for k in range(npair):
            rowv = sc[1 + buf_of[ncols_tuple[k]]]
            for ch in range(per_tile // chunk):
                base = base0 + ch * chunk
                pltpu.sync_copy(idxs[k].at[pl.ds(base, chunk)], idxv)
                pltpu.async_copy(tables[k].at[idxv], rowv, sem).wait()
                pltpu.sync_copy(rowv, outs[k].at[pl.ds(base, chunk)])

    out_type = [jax.ShapeDtypeStruct((nedge, nc), jnp.float32)
                for nc in ncols_tuple]
    return pl.kernel(body, mesh=_sc_mesh(), out_type=out_type,
                     scratch_types=scratch)


def _sc_gather(pairs):
    nedge = pairs[0][1].shape[0]
    ncols = tuple(int(t.shape[1]) for t, _ in pairs)
    k = _make_sc_gather(nedge, ncols)
    args = [t for t, _ in pairs] + [i for _, i in pairs]
    out = k(*args)
    return list(out) if isinstance(out, (tuple, list)) else [out]


def _unwrap(r):
    return r[0] if isinstance(r, (list, tuple)) else r


def _scatter_body(m_ref, d_ref, out_ref):
    i = pl.program_id(0)

    @pl.when(i == 0)
    def _init():
        out_ref[...] = jnp.zeros_like(out_ref)

    iota_n = lax.broadcasted_iota(jnp.int32, (1, _N), 1)
    oh = (d_ref[...] == iota_n).astype(jnp.bfloat16)
    m = m_ref[...]
    mh = m.astype(jnp.bfloat16)
    ml = (m - mh.astype(jnp.float32)).astype(jnp.bfloat16)
    dn = (((0,), (0,)), ((), ()))
    out_ref[...] += (
        lax.dot_general(oh, mh, dn, preferred_element_type=jnp.float32)
        + lax.dot_general(oh, ml, dn, preferred_element_type=jnp.float32))


def _scatter_stage(m, dst):
    nedge = m.shape[0]
    blk = 2048
    return pl.pallas_call(
        _scatter_body,
        grid=(nedge // blk,),
        in_specs=[
            pl.BlockSpec((blk, _MCOL), lambda i: (i, 0)),
            pl.BlockSpec((blk, 1), lambda i: (i, 0)),
        ],
        out_specs=pl.BlockSpec((_N, _MCOL), lambda i: (0, 0)),
        out_shape=jax.ShapeDtypeStruct((_N, _MCOL), jnp.float32),
    )(m, dst.reshape(nedge, 1))


_KEYS_PER_TILE = (_N * _N) // 16


@functools.lru_cache(maxsize=None)
def _make_sc_match():
    kpt = _KEYS_PER_TILE
    scratch = [
        pltpu.VMEM((kpt,), jnp.int32),
        pltpu.VMEM((4096,), jnp.int32),
        pltpu.VMEM((4096,), jnp.int32),
        pltpu.VMEM((_EL,), jnp.int32),
        pltpu.VMEM((_EL,), jnp.int32),
        pltpu.VMEM((_EL,), jnp.int32),
        pltpu.VMEM((16, 512), jnp.int32),
        pltpu.VMEM((512,), jnp.int32),
        pltpu.VMEM((512,), jnp.float32),
        pltpu.VMEM_SHARED((16, _EL), jnp.int32),
    ]

    def body(srcg, tgtg, srcl, tgtl, z_hbm, wsafe_hbm, maskf_hbm,
             table, eb1, eb2, lsrc, ltgt, wbuf, rbuf, wout, mout, stage):
        c = lax.axis_index("c")
        sid = lax.axis_index("s")
        lo = sid * kpt
        pltpu.sync_copy(z_hbm, table)
        for ch in range(_EG // 4096):
            cb = ch * 4096
            pltpu.sync_copy(srcg.at[pl.ds(cb, 4096)], eb1)
            pltpu.sync_copy(tgtg.at[pl.ds(cb, 4096)], eb2)

            def bfn(i, _, cb=cb):
                for u in range(4):
                    o = i * 64 + u * 16
                    key = eb1[pl.ds(o, 16)] * _N + eb2[pl.ds(o, 16)]
                    m = (key >= lo) & (key < lo + kpt)
                    idx = jnp.where(m, key - lo, 0)
                    ids = lax.iota(jnp.int32, 16) + (cb + o + 1)
                    plsc.store_scatter(table, [idx], ids, mask=m)
                return 0

            lax.fori_loop(0, 4096 // 64, bfn, 0)
        pltpu.sync_copy(srcl, lsrc)
        pltpu.sync_copy(tgtl, ltgt)
        pltpu.sync_copy(z_hbm.at[pl.ds(0, _EL)], wbuf)

        def lfn(i, _):
            for u in range(4):
                o = i * 64 + u * 16
                key = lsrc[pl.ds(o, 16)] * _N + ltgt[pl.ds(o, 16)]
                m = (key >= lo) & (key < lo + kpt)
                idx = jnp.where(m, key - lo, 0)
                vals = plsc.load_gather(table, [idx], mask=m)
                wbuf[pl.ds(o, 16)] = jnp.where(m, vals, 0)
            return 0

        lax.fori_loop(0, _EL // 64, lfn, 0)
        pltpu.sync_copy(wbuf, stage.at[sid])
        plsc.subcore_barrier()
        pltpu.sync_copy(stage.at[:, pl.ds(sid * 512, 512)], rbuf)

        def rfn(k, _):
            acc = rbuf[0, pl.ds(k * 16, 16)]
            for j in range(1, 16):
                acc = jnp.maximum(acc, rbuf[j, pl.ds(k * 16, 16)])
            wout[pl.ds(k * 16, 16)] = jnp.maximum(acc - 1, 0)
            mout[pl.ds(k * 16, 16)] = (acc > 0).astype(jnp.float32)
            return 0

        lax.fori_loop(0, 512 // 16, rfn, 0)

        @pl.when((sid < 8) == (c == 0))
        def _wr():
            pltpu.sync_copy(wout, wsafe_hbm.at[pl.ds(sid * 512, 512)])
            pltpu.sync_copy(mout, maskf_hbm.at[pl.ds(sid * 512, 512)])

    return pl.kernel(
        body, mesh=_sc_mesh(),
        out_type=[jax.ShapeDtypeStruct((_EL,), jnp.int32),
                  jax.ShapeDtypeStruct((_EL,), jnp.float32)],
        scratch_types=scratch,
        compiler_params=pltpu.CompilerParams(needs_layout_passes=False))


def _match_stage(src_g, tgt_g, src_l, tgt_l):
    z = jnp.zeros((_KEYS_PER_TILE,), jnp.int32)
    return _make_sc_match()(src_g, tgt_g, src_l, tgt_l, z)


def kernel(x, t, pos, edge_index_local, edge_index_global, edge_attr_global,
           batch, batch_edge_global, params):
    p = params
    src_l, tgt_l = edge_index_local[0], edge_index_local[1]
    src_g, tgt_g = edge_index_global[0], edge_index_global[1]

    pos8 = jnp.concatenate(
        [pos, jnp.zeros((_N, 4), jnp.float32), jnp.ones((_N, 1), jnp.float32)],
        axis=1)
    b2d = batch.astype(jnp.int32).reshape(_N, 1)
    be2d = batch_edge_global.astype(jnp.int32).reshape(_EG, 1)
    ea8 = jnp.pad(edge_attr_global, ((0, 0), (0, 3)))

    pos16, s, e_g = _embed(x, t, pos8, b2d, be2d, ea8, p)

    src_l2d = src_l.reshape(_EL, 1)
    tgt_l2d = tgt_l.reshape(_EL, 1)
    src_g2d = src_g.reshape(_EG, 1)
    tgt_g2d = tgt_g.reshape(_EG, 1)

    wsafe, maskf = _match_stage(src_g, tgt_g, src_l, tgt_l)
    sg_l, e_l = _sc_gather([(s, src_l), (e_g, wsafe)])
    m_l = _msg(sg_l, e_l, maskf.reshape(_EL, 1), tgt_l2d, src_l2d, pos16,
               p, _EL)
    part_l = _scatter_stage(m_l, tgt_l)
    s2 = _combine_s(s, part_l)

    sg_g = _sc_gather([(s2, src_g)])[0]
    ones_mask = jnp.ones((_EG, 1), jnp.float32)
    m_g = _msg(sg_g, e_g, ones_mask, tgt_g2d, src_g2d, pos16, p, _EG)
    part_g = _scatter_stage(m_g, tgt_g)

    wclbd = jnp.zeros((3 * _VDIM, 8), jnp.float32)
    for k in range(3):
        wclbd = wclbd.at[k * _VDIM:(k + 1) * _VDIM, k].set(p['W_cl'][:, 0])
    sh, atoms, c8 = _node_heads(s2, part_l, part_g, pos16, b2d, p, wclbd)

    shii, shjj = _sc_gather([(sh, tgt_g), (sh, src_g)])
    bonds = _bonds(shii, shjj, tgt_g2d, src_g2d, c8, e_g, p)

    coords = c8[:, :3]
    return coords, atoms, bonds


# speedup vs baseline: 1.0985x; 1.0954x over previous
import functools

import jax
import jax.numpy as jnp
from jax import lax
from jax.experimental import pallas as pl
from jax.experimental.pallas import tpu as pltpu

_N = 1024
_B = 32
_EL = 8192
_EG = 32768
_SDIM = 256
_VDIM = 64
_EDIM = 32
_MCOL = 512


def _silu(x):
    return x * jax.nn.sigmoid(x)


def _embed_node_body(x_ref, t_ref, pos8_ref, b_ref,
                     wtma_ref, btma_ref,
                     wam_ref, bam_ref, watm_ref, batm_ref,
                     pos16_ref, s_ref):
    f32 = jnp.float32
    iota_b = lax.broadcasted_iota(jnp.int32, (1, _B), 1)
    oh = (b_ref[...] == iota_b).astype(f32)
    pos8 = pos8_ref[...]
    sums = lax.dot_general(oh, pos8, (((0,), (0,)), ((), ())))
    cnt = jnp.maximum(sums[:, 7:8], 1.0)
    mean8 = sums / cnt
    posc8 = pos8 - jnp.dot(oh, mean8)
    pos16_ref[...] = jnp.concatenate(
        [posc8, jnp.zeros((_N, 120), f32)], axis=1)

    ta = t_ref[...] * wtma_ref[...] + btma_ref[...]
    tnode = jnp.dot(oh, ta)
    s0 = jnp.dot(x_ref[...], wam_ref[...]) + bam_ref[...] + tnode
    s_ref[...] = jnp.dot(s0, watm_ref[...]) + batm_ref[...]


def _embed_edge_body(t_ref, be_ref, ea8_ref,
                     wtmb_ref, btmb_ref, wbm8_ref, bbm_ref, wbtm_ref, bbtm_ref,
                     eg_ref):
    iota_b = lax.broadcasted_iota(jnp.int32, (1, _B), 1)
    ohe = (be_ref[...] == iota_b).astype(jnp.float32)
    tb = t_ref[...] * wtmb_ref[...] + btmb_ref[...]
    tedge = jnp.dot(ohe, tb)
    e0 = jnp.dot(ea8_ref[...], wbm8_ref[...]) + bbm_ref[...] + tedge
    eg = jnp.dot(e0, wbtm_ref[...]) + bbtm_ref[...]
    eg_ref[...] = jnp.concatenate(
        [eg, jnp.zeros((eg.shape[0], 128 - _EDIM), jnp.float32)], axis=1)


def _embed(x, t, pos8, b2d, be2d, ea8, p):
    pos16, s = pl.pallas_call(
        _embed_node_body,
        out_shape=(
            jax.ShapeDtypeStruct((_N, 128), jnp.float32),
            jax.ShapeDtypeStruct((_N, _SDIM), jnp.float32),
        ),
    )(x, t, pos8, b2d,
      p['W_tma'], p['b_tma'].reshape(1, -1),
      p['W_am'], p['b_am'].reshape(1, -1), p['W_atm'], p['b_atm'].reshape(1, -1))
    blk = 4096
    e_g = pl.pallas_call(
        _embed_edge_body,
        grid=(_EG // blk,),
        in_specs=[
            pl.BlockSpec((_B, 1), lambda i: (0, 0)),
            pl.BlockSpec((blk, 1), lambda i: (i, 0)),
            pl.BlockSpec((blk, 8), lambda i: (i, 0)),
            pl.BlockSpec((1, _EDIM), lambda i: (0, 0)),
            pl.BlockSpec((1, _EDIM), lambda i: (0, 0)),
            pl.BlockSpec((8, _EDIM), lambda i: (0, 0)),
            pl.BlockSpec((1, _EDIM), lambda i: (0, 0)),
            pl.BlockSpec((_EDIM, _EDIM), lambda i: (0, 0)),
            pl.BlockSpec((1, _EDIM), lambda i: (0, 0)),
        ],
        out_specs=pl.BlockSpec((blk, 128), lambda i: (i, 0)),
        out_shape=jax.ShapeDtypeStruct((_EG, 128), jnp.float32),
    )(t, be2d, ea8,
      p['W_tmb'], p['b_tmb'].reshape(1, -1),
      jnp.pad(p['W_bm'], ((0, 3), (0, 0))), p['b_bm'].reshape(1, -1),
      p['W_btm'], p['b_btm'].reshape(1, -1))
    return pos16, s, e_g


def _msg_body(sg_ref, el_ref, mk_ref, pt_ref, ps_ref,
              w1a_ref, w1b_ref, w1c_ref, b1_ref, wg2_ref, m_ref):
    r = pt_ref[...] - ps_ref[...]
    d2 = jnp.sum(r * r, axis=1, keepdims=True)
    d = jnp.sqrt(jnp.clip(d2, 1e-6, None))
    el = el_ref[...][:, :_EDIM] * mk_ref[...]
    pre = (jnp.dot(sg_ref[...], w1a_ref[...]) + jnp.dot(el, w1b_ref[...])
           + d * w1c_ref[...] + b1_ref[...])
    m = _silu(pre)
    gate = jnp.dot(m, wg2_ref[...])
    rn = r[:, 0:3] / d
    be = m.shape[0]
    ones = jnp.ones((be, 1), jnp.float32)
    zero = jnp.zeros((be, _MCOL - _SDIM - 3 * _VDIM - 1), jnp.float32)
    m_ref[...] = jnp.concatenate(
        [m, rn[:, 0:1] * gate, rn[:, 1:2] * gate, rn[:, 2:3] * gate, ones, zero],
        axis=1)


def _msg(sg, el, mk, pt, ps, p, nedge):
    blk = 2048
    grid = (nedge // blk,)
    w1a = p['W_g1'][:_SDIM]
    w1b = p['W_g1'][_SDIM:_SDIM + _EDIM]
    w1c = p['W_g1'][_SDIM + _EDIM].reshape(1, -1)
    return pl.pallas_call(
        _msg_body,
        grid=grid,
        in_specs=[
            pl.BlockSpec((blk, _SDIM), lambda i: (i, 0)),
            pl.BlockSpec((blk, 128), lambda i: (i, 0)),
            pl.BlockSpec((blk, 1), lambda i: (i, 0)),
            pl.BlockSpec((blk, 128), lambda i: (i, 0)),
            pl.BlockSpec((blk, 128), lambda i: (i, 0)),
            pl.BlockSpec((_SDIM, _SDIM), lambda i: (0, 0)),
            pl.BlockSpec((_EDIM, _SDIM), lambda i: (0, 0)),
            pl.BlockSpec((1, _SDIM), lambda i: (0, 0)),
            pl.BlockSpec((1, _SDIM), lambda i: (0, 0)),
            pl.BlockSpec((_SDIM, _VDIM), lambda i: (0, 0)),
        ],
        out_specs=pl.BlockSpec((blk, _MCOL), lambda i: (i, 0)),
        out_shape=jax.ShapeDtypeStruct((nedge, _MCOL), jnp.float32),
    )(sg, el, mk, pt, ps, w1a, w1b, w1c, p['b_g1'].reshape(1, -1), p['W_g2'])


def _comb_body(s_ref, pp_ref, s2_ref):
    ms = pp_ref[...]
    c = jnp.maximum(ms[:, _SDIM + 3 * _VDIM:_SDIM + 3 * _VDIM + 1], 1.0)
    s2_ref[...] = s_ref[...] + ms[:, :_SDIM] / c


def _combine_s(s, partials):
    return pl.pallas_call(
        _comb_body,
        out_shape=jax.ShapeDtypeStruct((_N, _SDIM), jnp.float32),
    )(s, partials)


def _node_body(s2_ref, pl_ref, pg_ref, pos16_ref, b_ref,
               wsh_ref, bsh_ref, wal_ref, bal_ref, wclbd_ref,
               sh_ref, atoms_ref, c16_ref):
    f32 = jnp.float32
    msg = pg_ref[...]
    cg = jnp.maximum(msg[:, _SDIM + 3 * _VDIM:_SDIM + 3 * _VDIM + 1], 1.0)
    s3 = s2_ref[...] + msg[:, :_SDIM] / cg
    msl = pl_ref[...]
    cl = jnp.maximum(msl[:, _SDIM + 3 * _VDIM:_SDIM + 3 * _VDIM + 1], 1.0)
    v3 = msl[:, _SDIM:_SDIM + 3 * _VDIM] / cl + msg[:, _SDIM:_SDIM + 3 * _VDIM] / cg

    sh = _silu(jnp.dot(s3, wsh_ref[...]) + bsh_ref[...])
    sh_ref[...] = sh
    atoms_ref[...] = jnp.dot(sh, wal_ref[...]) + bal_ref[...]

    coords0 = jnp.dot(v3, wclbd_ref[...])
    pos8 = pos16_ref[:, 0:8]
    iota_b = lax.broadcasted_iota(jnp.int32, (1, _B), 1)
    oh = (b_ref[...] == iota_b).astype(f32)
    ones = jnp.ones((_N, 1), f32)
    xcat = jnp.concatenate([coords0, pos8, ones], axis=1)
    sums = lax.dot_general(oh, xcat, (((0,), (0,)), ((), ())))
    cnt = jnp.maximum(sums[:, 16:17], 1.0)
    mean_c = sums[:, 0:8] / cnt
    mean_p = sums[:, 8:16] / cnt
    coords = (coords0 - jnp.dot(oh, mean_c)) + (pos8 - jnp.dot(oh, mean_p))
    c16_ref[...] = jnp.concatenate([coords, jnp.zeros((_N, 120), f32)], axis=1)


def _node_heads(s2, partials_l, partials_g, pos16, b2d, p, wclbd):
    return pl.pallas_call(
        _node_body,
        out_shape=(
            jax.ShapeDtypeStruct((_N, _SDIM), jnp.float32),
            jax.ShapeDtypeStruct((_N, 16), jnp.float32),
            jax.ShapeDtypeStruct((_N, 128), jnp.float32),
        ),
    )(s2, partials_l, partials_g, pos16, b2d,
      p['W_sh'], p['b_sh'].reshape(1, -1), p['W_al'], p['b_al'].reshape(1, -1),
      wclbd)


def _bond_body(shii_ref, shjj_ref, cii_ref, cjj_ref, eg_ref,
               whbm_ref, bhbm_ref, wb0a_ref, wb0c_ref, b0_ref,
               wb1_ref, bb1_ref, bonds_ref):
    f = (shii_ref[...] + shjj_ref[...]
         + jnp.dot(eg_ref[...][:, :_EDIM], whbm_ref[...]) + bhbm_ref[...])
    dc = cii_ref[...] - cjj_ref[...]
    de = jnp.sqrt(jnp.clip(jnp.sum(dc * dc, axis=1, keepdims=True), 1e-12, None))
    h = _silu(jnp.dot(f, wb0a_ref[...]) + de * wb0c_ref[...] + b0_ref[...])
    bonds_ref[...] = jnp.dot(h, wb1_ref[...]) + bb1_ref[...]


def _bonds(shii, shjj, cii, cjj, e_g, p):
    blk = 2048
    grid = (_EG // blk,)
    wb0a = p['W_b0'][:_SDIM]
    wb0c = p['W_b0'][_SDIM].reshape(1, -1)
    wb1 = jnp.pad(p['W_b1'], ((0, 0), (0, 3)))
    bb1 = jnp.pad(p['b_b1'], (0, 3)).reshape(1, -1)
    out = pl.pallas_call(
        _bond_body,
        grid=grid,
        in_specs=[
            pl.BlockSpec((blk, _SDIM), lambda i: (i, 0)),
            pl.BlockSpec((blk, _SDIM), lambda i: (i, 0)),
            pl.BlockSpec((blk, 128), lambda i: (i, 0)),
            pl.BlockSpec((blk, 128), lambda i: (i, 0)),
            pl.BlockSpec((blk, 128), lambda i: (i, 0)),
            pl.BlockSpec((_EDIM, _SDIM), lambda i: (0, 0)),
            pl.BlockSpec((1, _SDIM), lambda i: (0, 0)),
            pl.BlockSpec((_SDIM, _SDIM), lambda i: (0, 0)),
            pl.BlockSpec((1, _SDIM), lambda i: (0, 0)),
            pl.BlockSpec((1, _SDIM), lambda i: (0, 0)),
            pl.BlockSpec((_SDIM, 8), lambda i: (0, 0)),
            pl.BlockSpec((1, 8), lambda i: (0, 0)),
        ],
        out_specs=pl.BlockSpec((blk, 8), lambda i: (i, 0)),
        out_shape=jax.ShapeDtypeStruct((_EG, 8), jnp.float32),
    )(shii, shjj, cii, cjj, e_g,
      p['W_hbm'], p['b_hbm'].reshape(1, -1), wb0a, wb0c,
      p['b_b0'].reshape(1, -1), wb1, bb1)
    return out[:, :5]


from jax.experimental.pallas import tpu_sc as plsc

_NW = 32


def _sc_mesh():
    return plsc.VectorSubcoreMesh(core_axis_name="c", subcore_axis_name="s")


@functools.lru_cache(maxsize=None)
def _make_sc_gather(nedge, ncols_tuple):
    npair = len(ncols_tuple)
    per_tile = nedge // _NW
    chunk = min(256, per_tile)
    distinct = sorted(set(ncols_tuple))
    buf_of = {nc: i for i, nc in enumerate(distinct)}
    scratch = [pltpu.VMEM((chunk,), jnp.int32)]
    for nc in distinct:
        scratch.append(pltpu.VMEM((chunk, nc), jnp.float32))
    scratch.append(pltpu.SemaphoreType.DMA)

    def body(*refs):
        tables = refs[:npair]
        idxs = refs[npair:2 * npair]
        outs = refs[2 * npair:3 * npair]
        sc = refs[3 * npair:]
        idxv, sem = sc[0], sc[-1]
        wid = lax.axis_index("s") * 2 + lax.axis_index("c")
        base0 = wid * per_tile
        for k in range(npair):
            rowv = sc[1 + buf_of[ncols_tuple[k]]]
            for ch in range(per_tile // chunk):
                base = base0 + ch * chunk
                pltpu.sync_copy(idxs[k].at[pl.ds(base, chunk)], idxv)
                pltpu.async_copy(tables[k].at[idxv], rowv, sem).wait()
                pltpu.sync_copy(rowv, outs[k].at[pl.ds(base, chunk)])

    out_type = [jax.ShapeDtypeStruct((nedge, nc), jnp.float32)
                for nc in ncols_tuple]
    return pl.kernel(body, mesh=_sc_mesh(), out_type=out_type,
                     scratch_types=scratch)


def _sc_gather(pairs):
    nedge = pairs[0][1].shape[0]
    ncols = tuple(int(t.shape[1]) for t, _ in pairs)
    k = _make_sc_gather(nedge, ncols)
    args = [t for t, _ in pairs] + [i for _, i in pairs]
    out = k(*args)
    return list(out) if isinstance(out, (tuple, list)) else [out]


def _unwrap(r):
    return r[0] if isinstance(r, (list, tuple)) else r


def _scatter_body(m_ref, d_ref, out_ref):
    i = pl.program_id(0)

    @pl.when(i == 0)
    def _init():
        out_ref[...] = jnp.zeros_like(out_ref)

    iota_n = lax.broadcasted_iota(jnp.int32, (1, _N), 1)
    oh = (d_ref[...] == iota_n).astype(jnp.bfloat16)
    m = m_ref[...]
    mh = m.astype(jnp.bfloat16)
    ml = (m - mh.astype(jnp.float32)).astype(jnp.bfloat16)
    dn = (((0,), (0,)), ((), ()))
    out_ref[...] += (
        lax.dot_general(oh, mh, dn, preferred_element_type=jnp.float32)
        + lax.dot_general(oh, ml, dn, preferred_element_type=jnp.float32))


def _scatter_stage(m, dst):
    nedge = m.shape[0]
    blk = 2048
    return pl.pallas_call(
        _scatter_body,
        grid=(nedge // blk,),
        in_specs=[
            pl.BlockSpec((blk, _MCOL), lambda i: (i, 0)),
            pl.BlockSpec((blk, 1), lambda i: (i, 0)),
        ],
        out_specs=pl.BlockSpec((_N, _MCOL), lambda i: (0, 0)),
        out_shape=jax.ShapeDtypeStruct((_N, _MCOL), jnp.float32),
    )(m, dst.reshape(nedge, 1))


_KEYS_PER_TILE = (_N * _N) // 16


@functools.lru_cache(maxsize=None)
def _make_sc_match():
    kpt = _KEYS_PER_TILE
    scratch = [
        pltpu.VMEM((kpt,), jnp.int32),
        pltpu.VMEM((4096,), jnp.int32),
        pltpu.VMEM((4096,), jnp.int32),
        pltpu.VMEM((_EL,), jnp.int32),
        pltpu.VMEM((_EL,), jnp.int32),
        pltpu.VMEM((_EL,), jnp.int32),
        pltpu.VMEM((16, 512), jnp.int32),
        pltpu.VMEM((512,), jnp.int32),
        pltpu.VMEM((512,), jnp.float32),
        pltpu.VMEM_SHARED((16, _EL), jnp.int32),
    ]

    def body(srcg, tgtg, srcl, tgtl, z_hbm, wsafe_hbm, maskf_hbm,
             table, eb1, eb2, lsrc, ltgt, wbuf, rbuf, wout, mout, stage):
        c = lax.axis_index("c")
        sid = lax.axis_index("s")
        lo = sid * kpt
        pltpu.sync_copy(z_hbm, table)
        for ch in range(_EG // 4096):
            cb = ch * 4096
            pltpu.sync_copy(srcg.at[pl.ds(cb, 4096)], eb1)
            pltpu.sync_copy(tgtg.at[pl.ds(cb, 4096)], eb2)

            def bfn(i, _, cb=cb):
                for u in range(4):
                    o = i * 64 + u * 16
                    key = eb1[pl.ds(o, 16)] * _N + eb2[pl.ds(o, 16)]
                    m = (key >= lo) & (key < lo + kpt)
                    idx = jnp.where(m, key - lo, 0)
                    ids = lax.iota(jnp.int32, 16) + (cb + o + 1)
                    plsc.store_scatter(table, [idx], ids, mask=m)
                return 0

            lax.fori_loop(0, 4096 // 64, bfn, 0)
        pltpu.sync_copy(srcl, lsrc)
        pltpu.sync_copy(tgtl, ltgt)
        pltpu.sync_copy(z_hbm.at[pl.ds(0, _EL)], wbuf)

        def lfn(i, _):
            for u in range(4):
                o = i * 64 + u * 16
                key = lsrc[pl.ds(o, 16)] * _N + ltgt[pl.ds(o, 16)]
                m = (key >= lo) & (key < lo + kpt)
                idx = jnp.where(m, key - lo, 0)
                vals = plsc.load_gather(table, [idx], mask=m)
                wbuf[pl.ds(o, 16)] = jnp.where(m, vals, 0)
            return 0

        lax.fori_loop(0, _EL // 64, lfn, 0)
        pltpu.sync_copy(wbuf, stage.at[sid])
        plsc.subcore_barrier()
        pltpu.sync_copy(stage.at[:, pl.ds(sid * 512, 512)], rbuf)

        def rfn(k, _):
            acc = rbuf[0, pl.ds(k * 16, 16)]
            for j in range(1, 16):
                acc = jnp.maximum(acc, rbuf[j, pl.ds(k * 16, 16)])
            wout[pl.ds(k * 16, 16)] = jnp.maximum(acc - 1, 0)
            mout[pl.ds(k * 16, 16)] = (acc > 0).astype(jnp.float32)
            return 0

        lax.fori_loop(0, 512 // 16, rfn, 0)

        @pl.when((sid < 8) == (c == 0))
        def _wr():
            pltpu.sync_copy(wout, wsafe_hbm.at[pl.ds(sid * 512, 512)])
            pltpu.sync_copy(mout, maskf_hbm.at[pl.ds(sid * 512, 512)])

    return pl.kernel(
        body, mesh=_sc_mesh(),
        out_type=[jax.ShapeDtypeStruct((_EL,), jnp.int32),
                  jax.ShapeDtypeStruct((_EL,), jnp.float32)],
        scratch_types=scratch,
        compiler_params=pltpu.CompilerParams(needs_layout_passes=False))


def _match_stage(src_g, tgt_g, src_l, tgt_l):
    z = jnp.zeros((_KEYS_PER_TILE,), jnp.int32)
    return _make_sc_match()(src_g, tgt_g, src_l, tgt_l, z)


def kernel(x, t, pos, edge_index_local, edge_index_global, edge_attr_global,
           batch, batch_edge_global, params):
    p = params
    src_l, tgt_l = edge_index_local[0], edge_index_local[1]
    src_g, tgt_g = edge_index_global[0], edge_index_global[1]

    pos8 = jnp.concatenate(
        [pos, jnp.zeros((_N, 4), jnp.float32), jnp.ones((_N, 1), jnp.float32)],
        axis=1)
    b2d = batch.astype(jnp.int32).reshape(_N, 1)
    be2d = batch_edge_global.astype(jnp.int32).reshape(_EG, 1)
    ea8 = jnp.pad(edge_attr_global, ((0, 0), (0, 3)))

    pos16, s, e_g = _embed(x, t, pos8, b2d, be2d, ea8, p)

    wsafe, maskf = _match_stage(src_g, tgt_g, src_l, tgt_l)
    sg_l, e_l, pt_l, ps_l = _sc_gather(
        [(s, src_l), (e_g, wsafe), (pos16, tgt_l), (pos16, src_l)])
    m_l = _msg(sg_l, e_l, maskf.reshape(_EL, 1), pt_l, ps_l, p, _EL)
    part_l = _scatter_stage(m_l, tgt_l)
    s2 = _combine_s(s, part_l)

    sg_g, pt_g, ps_g = _sc_gather(
        [(s2, src_g), (pos16, tgt_g), (pos16, src_g)])
    ones_mask = jnp.ones((_EG, 1), jnp.float32)
    m_g = _msg(sg_g, e_g, ones_mask, pt_g, ps_g, p, _EG)
    part_g = _scatter_stage(m_g, tgt_g)

    wclbd = jnp.zeros((3 * _VDIM, 8), jnp.float32)
    for k in range(3):
        wclbd = wclbd.at[k * _VDIM:(k + 1) * _VDIM, k].set(p['W_cl'][:, 0])
    sh, atoms, c16 = _node_heads(s2, part_l, part_g, pos16, b2d, p, wclbd)

    shii, shjj, cii, cjj = _sc_gather(
        [(sh, tgt_g), (sh, src_g), (c16, tgt_g), (c16, src_g)])
    bonds = _bonds(shii, shjj, cii, cjj, e_g, p)

    coords = c16[:, :3]
    return coords, atoms, bonds
